# baseline (speedup 1.0000x reference)
import functools

import jax
import jax.numpy as jnp
from jax import lax
from jax.experimental import pallas as pl
from jax.experimental.pallas import tpu as pltpu
from jax.experimental.pallas import tpu_sc as plsc

N = 10000
E = 320000
D_IN = 128
D_CONV = 64
D_OUT = 128

NUM_CORES = 2
NUM_SUBCORES = 16
NW = NUM_CORES * NUM_SUBCORES

GCH = 192
SCH = 96
EDGES_PER_TILE = E // NW
N_GATHERS = -(-EDGES_PER_TILE // GCH)
EPT_PAD = N_GATHERS * GCH
N_SCATTERS = EPT_PAD // SCH

N_PAD = 10112
ROWS_PER_TILE = N_PAD // NUM_SUBCORES
CP = 128
_FULL = ROWS_PER_TILE // CP
_REM = ROWS_PER_TILE - _FULL * CP


def _mm_body(x_ref, w_ref, o_ref):
    o_ref[...] = jnp.dot(x_ref[...], w_ref[...],
                         preferred_element_type=jnp.float32)


def _matmul(x, w):
    return pl.pallas_call(
        _mm_body,
        out_shape=jax.ShapeDtypeStruct((x.shape[0], w.shape[1]), jnp.float32),
    )(x, w)


def _fused_body(p_ref, m_ref, w_ref, o_ref):
    h = p_ref[0, :N, :] + p_ref[1, :N, :] + m_ref[...]
    h = jnp.maximum(h, 0.0)
    o_ref[...] = jnp.dot(h, w_ref[...], preferred_element_type=jnp.float32)


def _fused_conv(p, m, w):
    return pl.pallas_call(
        _fused_body,
        out_shape=jax.ShapeDtypeStruct((N, w.shape[1]), jnp.float32),
    )(p, m, w)


def _final_body(p_ref, m_ref, state_ref, out_ref):
    st = p_ref[0, :N, :] + p_ref[1, :N, :] + m_ref[...]
    state_ref[...] = st
    out_ref[...] = jnp.sum(st, axis=0, keepdims=True)


def _final(p, m):
    return pl.pallas_call(
        _final_body,
        out_shape=(
            jax.ShapeDtypeStruct((N, D_OUT), jnp.float32),
            jax.ShapeDtypeStruct((1, D_OUT), jnp.float32),
        ),
    )(p, m)


def _make_seg_kernel(D):
    mesh = plsc.VectorSubcoreMesh(core_axis_name="c", subcore_axis_name="s")

    @functools.partial(
        pl.kernel,
        mesh=mesh,
        out_type=jax.ShapeDtypeStruct((NUM_CORES, N_PAD, D), jnp.float32),
        scratch_types=[
            pltpu.VMEM((EPT_PAD,), jnp.int32),
            pltpu.VMEM((N_SCATTERS, SCH), jnp.int32),
            pltpu.VMEM((GCH, D), jnp.float32),
            pltpu.VMEM_SHARED((N_PAD, D), jnp.float32),
            pltpu.SemaphoreType.DMA,
        ],
    )
    def seg(m_hbm, src_hbm, dst_hbm, out_hbm, src_v, dst_v, rows_v, agg_sh,
            sem):
        cid = lax.axis_index("c")
        sid = lax.axis_index("s")
        wid = sid * NUM_CORES + cid
        base = sid * ROWS_PER_TILE

        def _zero_row(i, carry):
            for c16 in range(D // 16):
                rows_v[i, pl.ds(c16 * 16, 16)] = jnp.zeros((16,), jnp.float32)
            return carry

        lax.fori_loop(0, CP, _zero_row, 0)
        for k in range(_FULL):
            pltpu.sync_copy(rows_v.at[pl.ds(0, CP)],
                            agg_sh.at[pl.ds(base + k * CP, CP)])
        pltpu.sync_copy(rows_v.at[pl.ds(0, _REM)],
                        agg_sh.at[pl.ds(base + _FULL * CP, _REM)])

        pltpu.sync_copy(src_hbm.at[wid], src_v)
        pltpu.sync_copy(dst_hbm.at[wid], dst_v)
        plsc.subcore_barrier()

        def _edges(g, carry):
            pltpu.async_copy(
                m_hbm.at[src_v.at[pl.ds(g * GCH, GCH)]], rows_v, sem
            ).wait()
            pltpu.sync_copy(rows_v.at[pl.ds(0, SCH)],
                            agg_sh.at[dst_v.at[2 * g]], add=True)
            pltpu.sync_copy(rows_v.at[pl.ds(SCH, SCH)],
                            agg_sh.at[dst_v.at[2 * g + 1]], add=True)
            return carry

        lax.fori_loop(0, N_GATHERS, _edges, 0)
        plsc.subcore_barrier()

        for k in range(_FULL):
            pltpu.sync_copy(agg_sh.at[pl.ds(base + k * CP, CP)],
                            rows_v.at[pl.ds(0, CP)])
            pltpu.sync_copy(rows_v.at[pl.ds(0, CP)],
                            out_hbm.at[cid, pl.ds(base + k * CP, CP)])
        pltpu.sync_copy(agg_sh.at[pl.ds(base + _FULL * CP, _REM)],
                        rows_v.at[pl.ds(0, _REM)])
        pltpu.sync_copy(rows_v.at[pl.ds(0, _REM)],
                        out_hbm.at[cid, pl.ds(base + _FULL * CP, _REM)])

    return seg


_seg = _make_seg_kernel(128)


def kernel(x, edge_index, gnn_ind, W1, Wh, W2):
    pad = NW * EPT_PAD - E
    src_p = jnp.concatenate(
        [edge_index[0], jnp.zeros((pad,), jnp.int32)]
    ).reshape(NW, EPT_PAD)
    dst_p = jnp.concatenate(
        [edge_index[1], jnp.full((pad,), N, jnp.int32)]
    ).reshape(NW, N_SCATTERS, SCH)

    W1p = jnp.pad(W1, ((0, 0), (0, 128 - D_CONV)))
    Whp = jnp.pad(Wh, ((0, 128 - D_CONV), (0, 128 - D_CONV)))
    W2p = jnp.pad(W2, ((0, 128 - D_CONV), (0, 0)))

    m1 = _matmul(x, W1p)
    p1 = _seg(m1, src_p, dst_p)
    m2 = _fused_conv(p1, m1, Whp)
    p2 = _seg(m2, src_p, dst_p)
    m3 = _fused_conv(p2, m2, W2p)
    p3 = _seg(m3, src_p, dst_p)
    state, out = _final(p3, m3)
    return (state, out)


# speedup vs baseline: 1.2494x; 1.2494x over previous
import functools

import jax
import jax.numpy as jnp
from jax import lax
from jax.experimental import pallas as pl
from jax.experimental.pallas import tpu as pltpu
from jax.experimental.pallas import tpu_sc as plsc

N = 10000
E = 320000
D_IN = 128
D_CONV = 64
D_OUT = 128

NUM_CORES = 2
NUM_SUBCORES = 16
NW = NUM_CORES * NUM_SUBCORES

CHUNK = 128
EDGES_PER_TILE = E // NW
N_CHUNKS = -(-EDGES_PER_TILE // CHUNK)
EPT_PAD = N_CHUNKS * CHUNK

N_PAD = 10112
ROWS_PER_TILE = N_PAD // NUM_SUBCORES
_FULL = ROWS_PER_TILE // CHUNK
_REM = ROWS_PER_TILE - _FULL * CHUNK


def _mm_body(x_ref, w_ref, o_ref):
    o_ref[...] = jnp.dot(x_ref[...], w_ref[...],
                         preferred_element_type=jnp.float32)


def _matmul(x, w):
    return pl.pallas_call(
        _mm_body,
        out_shape=jax.ShapeDtypeStruct((x.shape[0], w.shape[1]), jnp.float32),
    )(x, w)


def _fused_body(p_ref, m_ref, w_ref, o_ref):
    h = p_ref[0, :N, :] + p_ref[1, :N, :] + m_ref[...]
    h = jnp.maximum(h, 0.0)
    o_ref[...] = jnp.dot(h, w_ref[...], preferred_element_type=jnp.float32)


def _fused_conv(p, m, w):
    return pl.pallas_call(
        _fused_body,
        out_shape=jax.ShapeDtypeStruct((N, w.shape[1]), jnp.float32),
    )(p, m, w)


def _final_body(p_ref, m_ref, state_ref, out_ref):
    st = p_ref[0, :N, :] + p_ref[1, :N, :] + m_ref[...]
    state_ref[...] = st
    out_ref[...] = jnp.sum(st, axis=0, keepdims=True)


def _final(p, m):
    return pl.pallas_call(
        _final_body,
        out_shape=(
            jax.ShapeDtypeStruct((N, D_OUT), jnp.float32),
            jax.ShapeDtypeStruct((1, D_OUT), jnp.float32),
        ),
    )(p, m)


def _make_seg_kernel(D):
    mesh = plsc.VectorSubcoreMesh(core_axis_name="c", subcore_axis_name="s")

    @functools.partial(
        pl.kernel,
        mesh=mesh,
        out_type=jax.ShapeDtypeStruct((NUM_CORES, N_PAD, D), jnp.float32),
        scratch_types=[
            pltpu.VMEM((N_CHUNKS, CHUNK), jnp.int32),
            pltpu.VMEM((N_CHUNKS, CHUNK), jnp.int32),
            pltpu.VMEM((CHUNK, D), jnp.float32),
            pltpu.VMEM_SHARED((N_PAD, D), jnp.float32),
            pltpu.SemaphoreType.DMA,
        ],
    )
    def seg(m_hbm, src_hbm, dst_hbm, out_hbm, src_v, dst_v, rows_v, agg_sh,
            sem):
        cid = lax.axis_index("c")
        sid = lax.axis_index("s")
        wid = sid * NUM_CORES + cid
        base = sid * ROWS_PER_TILE

        def _zero_row(i, carry):
            for c16 in range(D // 16):
                rows_v[i, pl.ds(c16 * 16, 16)] = jnp.zeros((16,), jnp.float32)
            return carry

        lax.fori_loop(0, CHUNK, _zero_row, 0)
        for k in range(_FULL):
            pltpu.sync_copy(rows_v, agg_sh.at[pl.ds(base + k * CHUNK, CHUNK)])
        pltpu.sync_copy(rows_v.at[pl.ds(0, _REM)],
                        agg_sh.at[pl.ds(base + _FULL * CHUNK, _REM)])

        pltpu.sync_copy(src_hbm.at[wid], src_v)
        pltpu.sync_copy(dst_hbm.at[wid], dst_v)
        plsc.subcore_barrier()

        def _edges(j, carry):
            pltpu.async_copy(m_hbm.at[src_v.at[j]], rows_v, sem).wait()
            pltpu.sync_copy(rows_v, agg_sh.at[dst_v.at[j]], add=True)
            return carry

        lax.fori_loop(0, N_CHUNKS, _edges, 0)
        plsc.subcore_barrier()

        for k in range(_FULL):
            pltpu.sync_copy(agg_sh.at[pl.ds(base + k * CHUNK, CHUNK)], rows_v)
            pltpu.sync_copy(rows_v,
                            out_hbm.at[cid, pl.ds(base + k * CHUNK, CHUNK)])
        pltpu.sync_copy(agg_sh.at[pl.ds(base + _FULL * CHUNK, _REM)],
                        rows_v.at[pl.ds(0, _REM)])
        pltpu.sync_copy(rows_v.at[pl.ds(0, _REM)],
                        out_hbm.at[cid, pl.ds(base + _FULL * CHUNK, _REM)])

    return seg


_seg = _make_seg_kernel(128)


def kernel(x, edge_index, gnn_ind, W1, Wh, W2):
    pad = NW * EPT_PAD - E
    src_p = jnp.concatenate(
        [edge_index[0], jnp.zeros((pad,), jnp.int32)]
    ).reshape(NW, N_CHUNKS, CHUNK)
    dst_p = jnp.concatenate(
        [edge_index[1], jnp.full((pad,), N, jnp.int32)]
    ).reshape(NW, N_CHUNKS, CHUNK)

    W1p = jnp.pad(W1, ((0, 0), (0, 128 - D_CONV)))
    Whp = jnp.pad(Wh, ((0, 128 - D_CONV), (0, 128 - D_CONV)))
    W2p = jnp.pad(W2, ((0, 128 - D_CONV), (0, 0)))

    m1 = _matmul(x, W1p)
    p1 = _seg(m1, src_p, dst_p)
    m2 = _fused_conv(p1, m1, Whp)
    p2 = _seg(m2, src_p, dst_p)
    m3 = _fused_conv(p2, m2, W2p)
    p3 = _seg(m3, src_p, dst_p)
    state, out = _final(p3, m3)
    return (state, out)


# speedup vs baseline: 1.7097x; 1.3684x over previous
import functools

import jax
import jax.numpy as jnp
from jax import lax
from jax.experimental import pallas as pl
from jax.experimental.pallas import tpu as pltpu
from jax.experimental.pallas import tpu_sc as plsc

N = 10000
E = 320000
D_IN = 128
D_CONV = 64
D_OUT = 128

NUM_CORES = 2
NUM_SUBCORES = 16
NW = NUM_CORES * NUM_SUBCORES

CHUNK = 128
FAST_CID = 0
NCF = 98
NCS = 59
EDGES_F = NUM_SUBCORES * NCF * CHUNK
EDGES_S_PAD = NUM_SUBCORES * NCS * CHUNK

N_PAD = 10112
ROWS_PER_TILE = N_PAD // NUM_SUBCORES
_FULL = ROWS_PER_TILE // CHUNK
_REM = ROWS_PER_TILE - _FULL * CHUNK


def _mm_body(x_ref, w_ref, o_ref):
    o_ref[...] = jnp.dot(x_ref[...], w_ref[...],
                         preferred_element_type=jnp.float32)


def _matmul(x, w):
    return pl.pallas_call(
        _mm_body,
        out_shape=jax.ShapeDtypeStruct((x.shape[0], w.shape[1]), jnp.float32),
    )(x, w)


def _fused_body(p_ref, m_ref, w_ref, o_ref):
    h = p_ref[0, :N, :] + p_ref[1, :N, :] + m_ref[...]
    h = jnp.maximum(h, 0.0)
    o_ref[...] = jnp.dot(h, w_ref[...], preferred_element_type=jnp.float32)


def _fused_conv(p, m, w):
    return pl.pallas_call(
        _fused_body,
        out_shape=jax.ShapeDtypeStruct((N, w.shape[1]), jnp.float32),
    )(p, m, w)


def _final_body(p_ref, m_ref, state_ref, out_ref):
    st = p_ref[0, :N, :] + p_ref[1, :N, :] + m_ref[...]
    state_ref[...] = st
    out_ref[...] = jnp.sum(st, axis=0, keepdims=True)


def _final(p, m):
    return pl.pallas_call(
        _final_body,
        out_shape=(
            jax.ShapeDtypeStruct((N, D_OUT), jnp.float32),
            jax.ShapeDtypeStruct((1, D_OUT), jnp.float32),
        ),
    )(p, m)


def _make_seg_kernel(D):
    mesh = plsc.VectorSubcoreMesh(core_axis_name="c", subcore_axis_name="s")

    @functools.partial(
        pl.kernel,
        mesh=mesh,
        out_type=jax.ShapeDtypeStruct((NUM_CORES, N_PAD, D), jnp.float32),
        scratch_types=[
            pltpu.VMEM((NCF, CHUNK), jnp.int32),
            pltpu.VMEM((NCF, CHUNK), jnp.int32),
            pltpu.VMEM((CHUNK, D), jnp.float32),
            pltpu.VMEM_SHARED((N_PAD, D), jnp.float32),
            pltpu.SemaphoreType.DMA,
        ],
    )
    def seg(m_hbm, src_f, dst_f, src_s, dst_s, out_hbm, src_v, dst_v,
            rows_v, agg_sh, sem):
        cid = lax.axis_index("c")
        sid = lax.axis_index("s")
        base = sid * ROWS_PER_TILE

        def _zero_row(i, carry):
            for c16 in range(D // 16):
                rows_v[i, pl.ds(c16 * 16, 16)] = jnp.zeros((16,), jnp.float32)
            return carry

        lax.fori_loop(0, CHUNK, _zero_row, 0)
        for k in range(_FULL):
            pltpu.sync_copy(rows_v, agg_sh.at[pl.ds(base + k * CHUNK, CHUNK)])
        pltpu.sync_copy(rows_v.at[pl.ds(0, _REM)],
                        agg_sh.at[pl.ds(base + _FULL * CHUNK, _REM)])

        @pl.when(cid == FAST_CID)
        def _():
            pltpu.sync_copy(src_f.at[sid], src_v)
            pltpu.sync_copy(dst_f.at[sid], dst_v)

        @pl.when(cid != FAST_CID)
        def _():
            pltpu.sync_copy(src_s.at[sid], src_v.at[pl.ds(0, NCS)])
            pltpu.sync_copy(dst_s.at[sid], dst_v.at[pl.ds(0, NCS)])

        plsc.subcore_barrier()

        def _edges(j, carry):
            pltpu.async_copy(m_hbm.at[src_v.at[j]], rows_v, sem).wait()
            pltpu.sync_copy(rows_v, agg_sh.at[dst_v.at[j]], add=True)
            return carry

        lax.fori_loop(0, NCS, _edges, 0)

        @pl.when(cid == FAST_CID)
        def _():
            lax.fori_loop(NCS, NCF, _edges, 0)

        plsc.subcore_barrier()

        for k in range(_FULL):
            pltpu.sync_copy(agg_sh.at[pl.ds(base + k * CHUNK, CHUNK)], rows_v)
            pltpu.sync_copy(rows_v,
                            out_hbm.at[cid, pl.ds(base + k * CHUNK, CHUNK)])
        pltpu.sync_copy(agg_sh.at[pl.ds(base + _FULL * CHUNK, _REM)],
                        rows_v.at[pl.ds(0, _REM)])
        pltpu.sync_copy(rows_v.at[pl.ds(0, _REM)],
                        out_hbm.at[cid, pl.ds(base + _FULL * CHUNK, _REM)])

    return seg


_seg = _make_seg_kernel(128)


def kernel(x, edge_index, gnn_ind, W1, Wh, W2):
    pad = EDGES_F + EDGES_S_PAD - E
    src_f = edge_index[0][:EDGES_F].reshape(NUM_SUBCORES, NCF, CHUNK)
    dst_f = edge_index[1][:EDGES_F].reshape(NUM_SUBCORES, NCF, CHUNK)
    src_s = jnp.concatenate(
        [edge_index[0][EDGES_F:], jnp.zeros((pad,), jnp.int32)]
    ).reshape(NUM_SUBCORES, NCS, CHUNK)
    dst_s = jnp.concatenate(
        [edge_index[1][EDGES_F:], jnp.full((pad,), N, jnp.int32)]
    ).reshape(NUM_SUBCORES, NCS, CHUNK)

    W1p = jnp.pad(W1, ((0, 0), (0, 128 - D_CONV)))
    Whp = jnp.pad(Wh, ((0, 128 - D_CONV), (0, 128 - D_CONV)))
    W2p = jnp.pad(W2, ((0, 128 - D_CONV), (0, 0)))

    m1 = _matmul(x, W1p)
    p1 = _seg(m1, src_f, dst_f, src_s, dst_s)
    m2 = _fused_conv(p1, m1, Whp)
    p2 = _seg(m2, src_f, dst_f, src_s, dst_s)
    m3 = _fused_conv(p2, m2, W2p)
    p3 = _seg(m3, src_f, dst_f, src_s, dst_s)
    state, out = _final(p3, m3)
    return (state, out)
